# baseline (speedup 1.0000x reference)
import functools

import jax
import jax.numpy as jnp
from jax import lax
from jax.experimental import pallas as pl
from jax.experimental.pallas import tpu as pltpu
from jax.experimental.pallas import tpu_sc as plsc


def kernel(output, encoder_outputs, encoder_sequence_lengths):
    del output
    B, T, D = encoder_outputs.shape
    flat = encoder_outputs.reshape(B * T, D)
    lengths = jnp.asarray(encoder_sequence_lengths, jnp.int32)

    mesh = plsc.VectorSubcoreMesh(core_axis_name="c", subcore_axis_name="s")

    @functools.partial(
        pl.kernel,
        mesh=mesh,
        out_type=jax.ShapeDtypeStruct((B, D), jnp.float32),
        scratch_types=[
            pltpu.VMEM((B,), jnp.int32),
            pltpu.VMEM((B, D), jnp.float32),
            pltpu.SemaphoreType.DMA,
        ],
    )
    def gather_last(table_hbm, len_hbm, out_hbm, idx_v, rows_v, sem):
        wid = lax.axis_index("s") * 2 + lax.axis_index("c")

        @pl.when(wid == 0)
        def _():
            pltpu.sync_copy(len_hbm, idx_v)
            lens = idx_v[...]
            idx_v[...] = lens - 1 + lax.iota(jnp.int32, B) * T
            pltpu.async_copy(table_hbm.at[idx_v], rows_v, sem).wait()
            pltpu.sync_copy(rows_v, out_hbm)

    return gather_last(flat, lengths)


# speedup vs baseline: 1.0868x; 1.0868x over previous
import functools

import jax
import jax.numpy as jnp
from jax import lax
from jax.experimental import pallas as pl
from jax.experimental.pallas import tpu as pltpu
from jax.experimental.pallas import tpu_sc as plsc


def kernel(output, encoder_outputs, encoder_sequence_lengths):
    del output
    B, T, D = encoder_outputs.shape
    flat = encoder_outputs.reshape(B * T, D)
    lengths = jnp.asarray(encoder_sequence_lengths, jnp.int32)

    mesh = plsc.VectorSubcoreMesh(
        core_axis_name="c", subcore_axis_name="s", num_cores=1
    )

    @functools.partial(
        pl.kernel,
        mesh=mesh,
        out_type=jax.ShapeDtypeStruct((B, D), jnp.float32),
        scratch_types=[
            pltpu.VMEM((B,), jnp.int32),
            pltpu.VMEM((B, D), jnp.float32),
            pltpu.SemaphoreType.DMA,
        ],
    )
    def gather_last(table_hbm, len_hbm, out_hbm, idx_v, rows_v, sem):
        wid = lax.axis_index("s") * 2 + lax.axis_index("c")

        @pl.when(wid == 0)
        def _():
            pltpu.sync_copy(len_hbm, idx_v)
            lens = idx_v[...]
            idx_v[...] = lens - 1 + lax.iota(jnp.int32, B) * T
            pltpu.async_copy(table_hbm.at[idx_v], rows_v, sem).wait()
            pltpu.sync_copy(rows_v, out_hbm)

    return gather_last(flat, lengths)


# speedup vs baseline: 1.1194x; 1.0300x over previous
import functools

import jax
import jax.numpy as jnp
from jax.experimental import pallas as pl
from jax.experimental.pallas import tpu as pltpu
from jax.experimental.pallas import tpu_sc as plsc


def kernel(output, encoder_outputs, encoder_sequence_lengths):
    del output
    B, T, D = encoder_outputs.shape
    flat = encoder_outputs.reshape(B * T, D)
    lengths = jnp.asarray(encoder_sequence_lengths, jnp.int32)

    mesh = plsc.ScalarSubcoreMesh(axis_name="c", num_cores=1)

    @functools.partial(
        pl.kernel,
        mesh=mesh,
        out_type=jax.ShapeDtypeStruct((B, D), jnp.float32),
        scratch_types=[
            pltpu.SMEM((B,), jnp.int32),
            pltpu.SemaphoreType.DMA,
        ],
    )
    def gather_last(table_hbm, len_hbm, out_hbm, len_s, sem):
        pltpu.sync_copy(len_hbm, len_s)
        copies = []
        for b in range(B):
            idx = len_s[b] - 1 + b * T
            copies.append(
                pltpu.async_copy(
                    table_hbm.at[pl.ds(idx, 1)], out_hbm.at[pl.ds(b, 1)], sem
                )
            )
        for c in copies:
            c.wait()

    return gather_last(flat, lengths)
